# baseline (speedup 1.0000x reference)
import functools

import jax
import jax.numpy as jnp
from jax import lax
from jax.experimental import pallas as pl
from jax.experimental.pallas import tpu as pltpu
from jax.experimental.pallas import tpu_sc as plsc

N = 10000
E = 320000
D = 128
R = 8
NB = 4

NC = 2
NS = 16
C = 1000
CP = 1024
NCH = 10
CPS = NCH // NC
ET = E // NS
NV = ET // 16
CAP = 128
XP = 20480
BE = 2000
THR = CAP - 16


def _w_body(t_ref, w_ref):
    t = t_ref[...]
    ct = jnp.max(t)
    td = ct - t
    td = td / (jnp.max(td) + 1e-8)
    w = jnp.exp(-0.1 * td)
    w_ref[...] = w / (jnp.sum(w) + 1e-8)


_w_kernel = pl.pallas_call(
    _w_body,
    out_shape=jax.ShapeDtypeStruct((E // D, D), jnp.float32),
)


def _sc_body(meta_hbm, x_hbm, z_hbm,
             agg_out, tagg_out, cnt_out,
             mq, rows_v, trow_v,
             pk_buf, src_buf, loc_buf, cloc_buf, dstl_buf, w_buf, val_buf,
             agg_sp, tagg_sp, cnt_sp, sem, scat_sem):
    cid = lax.axis_index("c")
    sid = lax.axis_index("s")

    zrow = 10240 + sid * 16
    zero16f = jnp.zeros((16,), jnp.float32)
    zero16i = jnp.zeros((16,), jnp.int32)
    zrowv = jnp.full((16,), zrow, jnp.int32)

    one16f = jnp.ones((16,), jnp.float32)

    def reset_bufs():
        for t in range(CAP // 16):
            sl = pl.ds(t * 16, 16)
            pk_buf[sl] = zrowv
            w_buf[sl] = zero16f

    reset_bufs()

    s512 = pl.multiple_of(sid * 512, 512)
    s64 = pl.multiple_of(sid * 64, 64)
    s8k = pl.multiple_of(sid * 8192, 8192)

    def flush_core():
        for t in range(CAP // 16):
            sl = pl.ds(t * 16, 16)
            pk = pk_buf[sl]
            vsrc = pk & 16383
            dl = (pk >> 14) & 1023
            vtyp = pk >> 24
            src_buf[sl] = vsrc
            loc_buf[sl] = vtyp * CP + dl
            cloc_buf[sl] = dl * D + vtyp
            dstl_buf[sl] = dl
            val_buf[sl] = jnp.where(vsrc < N, one16f, zero16f)
        pltpu.async_copy(x_hbm.at[src_buf], rows_v, sem).wait()
        def rb(j, c):
            wv = plsc.load_gather(w_buf, [jnp.full((16,), j, jnp.int32)])
            for g in range(8):
                sl = pl.ds(g * 16, 16)
                trow_v[j, sl] = rows_v[j, sl] * wv
            return c
        lax.fori_loop(0, CAP, rb, 0, unroll=4)
        pltpu.async_copy(rows_v, agg_sp.at[loc_buf], scat_sem, add=True)
        pltpu.async_copy(val_buf, cnt_sp.at[cloc_buf], scat_sem, add=True)
        pltpu.async_copy(trow_v, tagg_sp.at[dstl_buf], scat_sem, add=True)
        reset_bufs()

    def wait_scatters():
        pltpu.make_async_copy(rows_v, agg_sp.at[loc_buf], scat_sem).wait()
        pltpu.make_async_copy(val_buf, cnt_sp.at[cloc_buf], scat_sem).wait()
        pltpu.make_async_copy(trow_v, tagg_sp.at[dstl_buf], scat_sem).wait()

    def flush():
        wait_scatters()
        flush_core()

    def chunk_body(i, _carry):
        k = cid * CPS + i
        c0 = k * C

        pltpu.sync_copy(x_hbm.at[pl.ds(10240, 512)],
                        agg_sp.at[pl.ds(s512, 512)])
        pltpu.sync_copy(x_hbm.at[pl.ds(10240, 64)],
                        tagg_sp.at[pl.ds(s64, 64)])
        pltpu.sync_copy(z_hbm, cnt_sp.at[pl.ds(s8k, 8192)])
        plsc.subcore_barrier()
        flush_core()

        def block_body(q, n_acc0):
            pltpu.sync_copy(meta_hbm.at[sid, q], mq)

            def vloop(v, n_acc):
                esl = pl.ds(v * 16, 16)
                vdst = mq[1, esl]
                vsrc = mq[0, esl]
                vtyp = mq[2, esl]
                vw = plsc.bitcast(mq[3, esl], jnp.float32)
                dl = vdst - c0
                m = (dl >= 0) & (dl < C)
                pk = vsrc + (dl << 14) + (vtyp << 24)
                cum = plsc.cumsum(m.astype(jnp.int32))
                pos = n_acc + cum - 1
                plsc.store_scatter(pk_buf, [pos], pk, mask=m)
                plsc.store_scatter(w_buf, [pos], vw, mask=m)
                n_acc = n_acc + jnp.max(cum)

                @pl.when(n_acc > THR)
                def _():
                    flush()

                return jnp.where(n_acc > THR, 0, n_acc)

            return lax.fori_loop(0, BE // 16, vloop, n_acc0, unroll=2)

        lax.fori_loop(0, ET // BE, block_body, 0)
        flush()
        wait_scatters()

        plsc.subcore_barrier()
        pltpu.sync_copy(agg_sp.at[pl.ds(s512, 512)],
                        agg_out.at[k, pl.ds(s512, 512)])
        pltpu.sync_copy(tagg_sp.at[pl.ds(s64, 64)],
                        tagg_out.at[k, pl.ds(s64, 64)])
        pltpu.sync_copy(cnt_sp.at[pl.ds(s8k, 8192)],
                        cnt_out.at[k, pl.ds(s8k, 8192)])
        plsc.subcore_barrier()
        return 0

    lax.fori_loop(0, CPS, chunk_body, 0)


@functools.cache
def _get_sc_kernel():
    return functools.partial(
        pl.kernel,
        out_type=(
            jax.ShapeDtypeStruct((NCH, R * CP, D), jnp.float32),
            jax.ShapeDtypeStruct((NCH, CP, D), jnp.float32),
            jax.ShapeDtypeStruct((NCH, CP * D), jnp.float32),
        ),
        mesh=plsc.VectorSubcoreMesh(core_axis_name="c", subcore_axis_name="s"),
        compiler_params=pltpu.CompilerParams(needs_layout_passes=False, use_tc_tiling_on_sc=False),
        scratch_types=[
            pltpu.VMEM((4, BE), jnp.int32),
            pltpu.VMEM((CAP, D), jnp.float32),
            pltpu.VMEM((CAP, D), jnp.float32),
            pltpu.VMEM((CAP,), jnp.int32),
            pltpu.VMEM((CAP,), jnp.int32),
            pltpu.VMEM((CAP,), jnp.int32),
            pltpu.VMEM((CAP,), jnp.int32),
            pltpu.VMEM((CAP,), jnp.int32),
            pltpu.VMEM((CAP,), jnp.float32),
            pltpu.VMEM((CAP,), jnp.float32),
            pltpu.VMEM_SHARED((R * CP, D), jnp.float32),
            pltpu.VMEM_SHARED((CP, D), jnp.float32),
            pltpu.VMEM_SHARED((CP * D,), jnp.float32),
            pltpu.SemaphoreType.DMA,
            pltpu.SemaphoreType.DMA,
        ],
    )(_sc_body)


def _dense_body(agg_ref, cnt_ref, tagg_ref, x_ref, comp_ref, bases_ref,
                root_ref, bias_ref, wtpT_ref, btp_ref, gamma_ref, beta_ref,
                out_ref):
    xb = x_ref[...]
    cnt = cnt_ref[0]
    acc = jnp.dot(xb, root_ref[...], preferred_element_type=jnp.float32)
    for r in range(R):
        a = agg_ref[0, pl.ds(r * CP, C), :]
        rec = 1.0 / jnp.maximum(cnt[0:C, r:r + 1], 1.0)
        wr = comp_ref[r, 0] * bases_ref[0]
        for b in range(1, NB):
            wr = wr + comp_ref[r, b] * bases_ref[b]
        acc = acc + jnp.dot(a * rec, wr, preferred_element_type=jnp.float32)
    deg = jnp.sum(cnt[0:C, 0:R], axis=1, keepdims=True)
    t = tagg_ref[0, 0:C, :] / jnp.maximum(deg, 1.0)
    out = acc + bias_ref[...] + jnp.dot(t, wtpT_ref[...],
                                        preferred_element_type=jnp.float32)
    out = out + btp_ref[...]
    out = jnp.maximum(out, 0.0)
    mu = jnp.mean(out, axis=1, keepdims=True)
    var = jnp.mean((out - mu) ** 2, axis=1, keepdims=True)
    out_ref[...] = (out - mu) * lax.rsqrt(var + 1e-5) * gamma_ref[...] \
        + beta_ref[...]


_dense_kernel = pl.pallas_call(
    _dense_body,
    grid=(NCH,),
    in_specs=[
        pl.BlockSpec((1, R * CP, D), lambda k: (k, 0, 0)),
        pl.BlockSpec((1, CP, D), lambda k: (k, 0, 0)),
        pl.BlockSpec((1, CP, D), lambda k: (k, 0, 0)),
        pl.BlockSpec((C, D), lambda k: (k, 0)),
        pl.BlockSpec((R, NB), lambda k: (0, 0)),
        pl.BlockSpec((NB, D, D), lambda k: (0, 0, 0)),
        pl.BlockSpec((D, D), lambda k: (0, 0)),
        pl.BlockSpec((1, D), lambda k: (0, 0)),
        pl.BlockSpec((D, D), lambda k: (0, 0)),
        pl.BlockSpec((1, D), lambda k: (0, 0)),
        pl.BlockSpec((1, D), lambda k: (0, 0)),
        pl.BlockSpec((1, D), lambda k: (0, 0)),
    ],
    out_specs=pl.BlockSpec((C, D), lambda k: (k, 0)),
    out_shape=jax.ShapeDtypeStruct((N, D), jnp.float32),
)


def kernel(x, edge_index, edge_type, edge_time, comp, bases, root, bias,
           W_tp, b_tp, gamma, beta):
    w = _w_kernel(edge_time.reshape(E // D, D)).reshape(E)
    w_i = jax.lax.bitcast_convert_type(w, jnp.int32)
    meta = jnp.stack([edge_index[0].reshape(NS, ET // BE, BE),
                      edge_index[1].reshape(NS, ET // BE, BE),
                      edge_type.reshape(NS, ET // BE, BE),
                      w_i.reshape(NS, ET // BE, BE)], axis=2)
    xp = jnp.zeros((XP, D), jnp.float32).at[:N].set(x)
    z = jnp.zeros((8192,), jnp.float32)

    agg_raw, tagg_raw, cnt_raw = _get_sc_kernel()(meta, xp, z)
    cnt3 = cnt_raw.reshape(NCH, CP, D)

    out = _dense_kernel(agg_raw, cnt3, tagg_raw, x, comp, bases, root,
                        bias.reshape(1, D), W_tp.T, b_tp.reshape(1, D),
                        gamma.reshape(1, D), beta.reshape(1, D))
    return out


# speedup vs baseline: 1.3541x; 1.3541x over previous
import functools

import jax
import jax.numpy as jnp
from jax import lax
from jax.experimental import pallas as pl
from jax.experimental.pallas import tpu as pltpu
from jax.experimental.pallas import tpu_sc as plsc

N = 10000
E = 320000
D = 128
R = 8
NB = 4

NC = 2
NS = 16
C = 1000
CP = 1024
NCH = 10
CPS = NCH // NC
ET = E // NS
NV = ET // 16
CAP = 128
XP = 20480
BE = 4000
THR = CAP - 16


def _w_body(t_ref, w_ref):
    t = t_ref[...]
    ct = jnp.max(t)
    td = ct - t
    td = td / (jnp.max(td) + 1e-8)
    w = jnp.exp(-0.1 * td)
    w_ref[...] = w / (jnp.sum(w) + 1e-8)


_w_kernel = pl.pallas_call(
    _w_body,
    out_shape=jax.ShapeDtypeStruct((E // D, D), jnp.float32),
)


def _sc_body(meta_hbm, x_hbm, z_hbm,
             agg_out, tagg_out, cnt_out,
             mq, rows_v,
             pk_buf, src_buf, loc_buf, cloc_buf, dstl_buf, w_buf, val_buf,
             agg_sp, tagg_sp, cnt_sp, sem):
    cid = lax.axis_index("c")
    sid = lax.axis_index("s")

    zrow = 10240 + sid * 16
    zero16f = jnp.zeros((16,), jnp.float32)
    zero16i = jnp.zeros((16,), jnp.int32)
    zrowv = jnp.full((16,), zrow, jnp.int32)

    one16f = jnp.ones((16,), jnp.float32)

    def reset_bufs():
        for t in range(CAP // 16):
            sl = pl.ds(t * 16, 16)
            pk_buf[sl] = zrowv
            w_buf[sl] = zero16f

    reset_bufs()

    s512 = pl.multiple_of(sid * 512, 512)
    s64 = pl.multiple_of(sid * 64, 64)
    s8k = pl.multiple_of(sid * 8192, 8192)

    def flush():
        for t in range(CAP // 16):
            sl = pl.ds(t * 16, 16)
            pk = pk_buf[sl]
            vsrc = pk & 16383
            dl = (pk >> 14) & 1023
            vtyp = pk >> 24
            src_buf[sl] = vsrc
            loc_buf[sl] = vtyp * CP + dl
            cloc_buf[sl] = dl * D + vtyp
            dstl_buf[sl] = dl
            val_buf[sl] = jnp.where(vsrc < N, one16f, zero16f)
        pltpu.async_copy(x_hbm.at[src_buf], rows_v, sem).wait()
        pltpu.sync_copy(rows_v, agg_sp.at[loc_buf], add=True)
        pltpu.sync_copy(val_buf, cnt_sp.at[cloc_buf], add=True)
        def rb(j, c):
            wv = plsc.load_gather(w_buf, [jnp.full((16,), j, jnp.int32)])
            for g in range(8):
                sl = pl.ds(g * 16, 16)
                rows_v[j, sl] = rows_v[j, sl] * wv
            return c
        lax.fori_loop(0, CAP, rb, 0, unroll=4)
        pltpu.sync_copy(rows_v, tagg_sp.at[dstl_buf], add=True)
        reset_bufs()

    def chunk_body(i, _carry):
        k = cid * CPS + i
        c0 = k * C

        pltpu.sync_copy(x_hbm.at[pl.ds(10240, 512)],
                        agg_sp.at[pl.ds(s512, 512)])
        pltpu.sync_copy(x_hbm.at[pl.ds(10240, 64)],
                        tagg_sp.at[pl.ds(s64, 64)])
        pltpu.sync_copy(z_hbm, cnt_sp.at[pl.ds(s8k, 8192)])
        plsc.subcore_barrier()

        def block_body(q, n_acc0):
            pltpu.sync_copy(meta_hbm.at[sid, q], mq)

            def vloop(v, n_acc):
                esl = pl.ds(v * 16, 16)
                vdst = mq[1, esl]
                vsrc = mq[0, esl]
                vtyp = mq[2, esl]
                vw = plsc.bitcast(mq[3, esl], jnp.float32)
                dl = vdst - c0
                m = (dl >= 0) & (dl < C)
                pk = vsrc + (dl << 14) + (vtyp << 24)
                cum = plsc.cumsum(m.astype(jnp.int32))
                pos = n_acc + cum - 1
                plsc.store_scatter(pk_buf, [pos], pk, mask=m)
                plsc.store_scatter(w_buf, [pos], vw, mask=m)
                n_acc = n_acc + jnp.max(cum)

                @pl.when(n_acc > THR)
                def _():
                    flush()

                return jnp.where(n_acc > THR, 0, n_acc)

            return lax.fori_loop(0, BE // 16, vloop, n_acc0, unroll=2)

        lax.fori_loop(0, ET // BE, block_body, 0)
        flush()

        plsc.subcore_barrier()
        pltpu.sync_copy(agg_sp.at[pl.ds(s512, 512)],
                        agg_out.at[k, pl.ds(s512, 512)])
        pltpu.sync_copy(tagg_sp.at[pl.ds(s64, 64)],
                        tagg_out.at[k, pl.ds(s64, 64)])
        pltpu.sync_copy(cnt_sp.at[pl.ds(s8k, 8192)],
                        cnt_out.at[k, pl.ds(s8k, 8192)])
        plsc.subcore_barrier()
        return 0

    lax.fori_loop(0, CPS, chunk_body, 0)


@functools.cache
def _get_sc_kernel():
    return functools.partial(
        pl.kernel,
        out_type=(
            jax.ShapeDtypeStruct((NCH, R * CP, D), jnp.float32),
            jax.ShapeDtypeStruct((NCH, CP, D), jnp.float32),
            jax.ShapeDtypeStruct((NCH, CP * D), jnp.float32),
        ),
        mesh=plsc.VectorSubcoreMesh(core_axis_name="c", subcore_axis_name="s"),
        compiler_params=pltpu.CompilerParams(needs_layout_passes=False, use_tc_tiling_on_sc=False),
        scratch_types=[
            pltpu.VMEM((4, BE), jnp.int32),
            pltpu.VMEM((CAP, D), jnp.float32),
            pltpu.VMEM((CAP,), jnp.int32),
            pltpu.VMEM((CAP,), jnp.int32),
            pltpu.VMEM((CAP,), jnp.int32),
            pltpu.VMEM((CAP,), jnp.int32),
            pltpu.VMEM((CAP,), jnp.int32),
            pltpu.VMEM((CAP,), jnp.float32),
            pltpu.VMEM((CAP,), jnp.float32),
            pltpu.VMEM_SHARED((R * CP, D), jnp.float32),
            pltpu.VMEM_SHARED((CP, D), jnp.float32),
            pltpu.VMEM_SHARED((CP * D,), jnp.float32),
            pltpu.SemaphoreType.DMA,
        ],
    )(_sc_body)


def _dense_body(agg_ref, cnt_ref, tagg_ref, x_ref, comp_ref, bases_ref,
                root_ref, bias_ref, wtpT_ref, btp_ref, gamma_ref, beta_ref,
                out_ref):
    xb = x_ref[...]
    cnt = cnt_ref[0]
    acc = jnp.dot(xb, root_ref[...], preferred_element_type=jnp.float32)
    for r in range(R):
        a = agg_ref[0, pl.ds(r * CP, C), :]
        rec = 1.0 / jnp.maximum(cnt[0:C, r:r + 1], 1.0)
        wr = comp_ref[r, 0] * bases_ref[0]
        for b in range(1, NB):
            wr = wr + comp_ref[r, b] * bases_ref[b]
        acc = acc + jnp.dot(a * rec, wr, preferred_element_type=jnp.float32)
    deg = jnp.sum(cnt[0:C, 0:R], axis=1, keepdims=True)
    t = tagg_ref[0, 0:C, :] / jnp.maximum(deg, 1.0)
    out = acc + bias_ref[...] + jnp.dot(t, wtpT_ref[...],
                                        preferred_element_type=jnp.float32)
    out = out + btp_ref[...]
    out = jnp.maximum(out, 0.0)
    mu = jnp.mean(out, axis=1, keepdims=True)
    var = jnp.mean((out - mu) ** 2, axis=1, keepdims=True)
    out_ref[...] = (out - mu) * lax.rsqrt(var + 1e-5) * gamma_ref[...] \
        + beta_ref[...]


_dense_kernel = pl.pallas_call(
    _dense_body,
    grid=(NCH,),
    in_specs=[
        pl.BlockSpec((1, R * CP, D), lambda k: (k, 0, 0)),
        pl.BlockSpec((1, CP, D), lambda k: (k, 0, 0)),
        pl.BlockSpec((1, CP, D), lambda k: (k, 0, 0)),
        pl.BlockSpec((C, D), lambda k: (k, 0)),
        pl.BlockSpec((R, NB), lambda k: (0, 0)),
        pl.BlockSpec((NB, D, D), lambda k: (0, 0, 0)),
        pl.BlockSpec((D, D), lambda k: (0, 0)),
        pl.BlockSpec((1, D), lambda k: (0, 0)),
        pl.BlockSpec((D, D), lambda k: (0, 0)),
        pl.BlockSpec((1, D), lambda k: (0, 0)),
        pl.BlockSpec((1, D), lambda k: (0, 0)),
        pl.BlockSpec((1, D), lambda k: (0, 0)),
    ],
    out_specs=pl.BlockSpec((C, D), lambda k: (k, 0)),
    out_shape=jax.ShapeDtypeStruct((N, D), jnp.float32),
)


def kernel(x, edge_index, edge_type, edge_time, comp, bases, root, bias,
           W_tp, b_tp, gamma, beta):
    w = _w_kernel(edge_time.reshape(E // D, D)).reshape(E)
    w_i = jax.lax.bitcast_convert_type(w, jnp.int32)
    meta = jnp.stack([edge_index[0].reshape(NS, ET // BE, BE),
                      edge_index[1].reshape(NS, ET // BE, BE),
                      edge_type.reshape(NS, ET // BE, BE),
                      w_i.reshape(NS, ET // BE, BE)], axis=2)
    xp = jnp.zeros((XP, D), jnp.float32).at[:N].set(x)
    z = jnp.zeros((8192,), jnp.float32)

    agg_raw, tagg_raw, cnt_raw = _get_sc_kernel()(meta, xp, z)
    cnt3 = cnt_raw.reshape(NCH, CP, D)

    out = _dense_kernel(agg_raw, cnt3, tagg_raw, x, comp, bases, root,
                        bias.reshape(1, D), W_tp.T, b_tp.reshape(1, D),
                        gamma.reshape(1, D), beta.reshape(1, D))
    return out


# speedup vs baseline: 1.4391x; 1.0628x over previous
import functools

import jax
import jax.numpy as jnp
from jax import lax
from jax.experimental import pallas as pl
from jax.experimental.pallas import tpu as pltpu
from jax.experimental.pallas import tpu_sc as plsc

N = 10000
E = 320000
D = 128
R = 8
NB = 4

NC = 2
NS = 16
C = 1000
CP = 1024
NCH = 10
CPS = NCH // NC
ET = E // NS
NV = ET // 16
CAP = 128
XP = 20480
BE = 4000
THR = CAP - 16


def _w_body(t_ref, w_ref):
    t = t_ref[...]
    ct = jnp.max(t)
    td = ct - t
    td = td / (jnp.max(td) + 1e-8)
    w = jnp.exp(-0.1 * td)
    w_ref[...] = w / (jnp.sum(w) + 1e-8)


_w_kernel = pl.pallas_call(
    _w_body,
    out_shape=jax.ShapeDtypeStruct((E // D, D), jnp.float32),
)


def _sc_body(meta_hbm, x_hbm, z_hbm,
             agg_out, tagg_out, cnt_out,
             mq, rows_v,
             pk_buf, src_buf, loc_buf, cloc_buf, dstl_buf, w_buf, val_buf,
             agg_sp, tagg_sp, cnt_sp, sem):
    cid = lax.axis_index("c")
    sid = lax.axis_index("s")

    zrow = 10240 + sid * 16
    zero16f = jnp.zeros((16,), jnp.float32)
    zero16i = jnp.zeros((16,), jnp.int32)
    zrowv = jnp.full((16,), zrow, jnp.int32)

    one16f = jnp.ones((16,), jnp.float32)

    def reset_bufs():
        for t in range(CAP // 16):
            sl = pl.ds(t * 16, 16)
            pk_buf[sl] = zrowv
            w_buf[sl] = zero16f

    reset_bufs()

    s512 = pl.multiple_of(sid * 512, 512)
    s64 = pl.multiple_of(sid * 64, 64)
    s8k = pl.multiple_of(sid * 8192, 8192)

    def flush():
        for t in range(CAP // 16):
            sl = pl.ds(t * 16, 16)
            pk = pk_buf[sl]
            vsrc = pk & 16383
            dl = (pk >> 14) & 1023
            vtyp = pk >> 24
            src_buf[sl] = vsrc
            loc_buf[sl] = vtyp * CP + dl
            cloc_buf[sl] = dl * D + vtyp
            dstl_buf[sl] = dl
            val_buf[sl] = jnp.where(vsrc < N, one16f, zero16f)
        pltpu.async_copy(x_hbm.at[src_buf], rows_v, sem).wait()
        pltpu.sync_copy(rows_v, agg_sp.at[loc_buf], add=True)
        pltpu.sync_copy(val_buf, cnt_sp.at[cloc_buf], add=True)
        def rb(j, c):
            wv = plsc.load_gather(w_buf, [jnp.full((16,), j, jnp.int32)])
            for g in range(8):
                sl = pl.ds(g * 16, 16)
                rows_v[j, sl] = rows_v[j, sl] * wv
            return c
        lax.fori_loop(0, CAP, rb, 0, unroll=4)
        pltpu.sync_copy(rows_v, tagg_sp.at[dstl_buf], add=True)
        reset_bufs()

    def chunk_body(i, _carry):
        k = cid * CPS + i
        c0 = k * C

        pltpu.sync_copy(x_hbm.at[pl.ds(10240, 512)],
                        agg_sp.at[pl.ds(s512, 512)])
        pltpu.sync_copy(x_hbm.at[pl.ds(10240, 64)],
                        tagg_sp.at[pl.ds(s64, 64)])
        pltpu.sync_copy(z_hbm, cnt_sp.at[pl.ds(s8k, 8192)])
        plsc.subcore_barrier()

        def block_body(q, n_acc0):
            pltpu.sync_copy(meta_hbm.at[sid, q], mq)

            def vloop(v, n_acc):
                esl = pl.ds(v * 16, 16)
                vdst = mq[1, esl]
                vsrc = mq[0, esl]
                vtyp = mq[2, esl]
                vw = plsc.bitcast(mq[3, esl], jnp.float32)
                dl = vdst - c0
                m = (dl >= 0) & (dl < C)
                pk = vsrc + (dl << 14) + (vtyp << 24)
                cum = plsc.cumsum(m.astype(jnp.int32))
                pos = n_acc + cum - 1
                plsc.store_scatter(pk_buf, [pos], pk, mask=m)
                plsc.store_scatter(w_buf, [pos], vw, mask=m)
                n_acc = n_acc + plsc.all_reduce_population_count(m)[0]

                @pl.when(n_acc > THR)
                def _():
                    flush()

                return jnp.where(n_acc > THR, 0, n_acc)

            return lax.fori_loop(0, BE // 16, vloop, n_acc0, unroll=2)

        lax.fori_loop(0, ET // BE, block_body, 0)
        flush()

        plsc.subcore_barrier()
        pltpu.sync_copy(agg_sp.at[pl.ds(s512, 512)],
                        agg_out.at[k, pl.ds(s512, 512)])
        pltpu.sync_copy(tagg_sp.at[pl.ds(s64, 64)],
                        tagg_out.at[k, pl.ds(s64, 64)])
        pltpu.sync_copy(cnt_sp.at[pl.ds(s8k, 8192)],
                        cnt_out.at[k, pl.ds(s8k, 8192)])
        plsc.subcore_barrier()
        return 0

    lax.fori_loop(0, CPS, chunk_body, 0)


@functools.cache
def _get_sc_kernel():
    return functools.partial(
        pl.kernel,
        out_type=(
            jax.ShapeDtypeStruct((NCH, R * CP, D), jnp.float32),
            jax.ShapeDtypeStruct((NCH, CP, D), jnp.float32),
            jax.ShapeDtypeStruct((NCH, CP * D), jnp.float32),
        ),
        mesh=plsc.VectorSubcoreMesh(core_axis_name="c", subcore_axis_name="s"),
        compiler_params=pltpu.CompilerParams(needs_layout_passes=False, use_tc_tiling_on_sc=False),
        scratch_types=[
            pltpu.VMEM((4, BE), jnp.int32),
            pltpu.VMEM((CAP, D), jnp.float32),
            pltpu.VMEM((CAP,), jnp.int32),
            pltpu.VMEM((CAP,), jnp.int32),
            pltpu.VMEM((CAP,), jnp.int32),
            pltpu.VMEM((CAP,), jnp.int32),
            pltpu.VMEM((CAP,), jnp.int32),
            pltpu.VMEM((CAP,), jnp.float32),
            pltpu.VMEM((CAP,), jnp.float32),
            pltpu.VMEM_SHARED((R * CP, D), jnp.float32),
            pltpu.VMEM_SHARED((CP, D), jnp.float32),
            pltpu.VMEM_SHARED((CP * D,), jnp.float32),
            pltpu.SemaphoreType.DMA,
        ],
    )(_sc_body)


def _dense_body(agg_ref, cnt_ref, tagg_ref, x_ref, comp_ref, bases_ref,
                root_ref, bias_ref, wtpT_ref, btp_ref, gamma_ref, beta_ref,
                out_ref):
    xb = x_ref[...]
    cnt = cnt_ref[0]
    acc = jnp.dot(xb, root_ref[...], preferred_element_type=jnp.float32)
    for r in range(R):
        a = agg_ref[0, pl.ds(r * CP, C), :]
        rec = 1.0 / jnp.maximum(cnt[0:C, r:r + 1], 1.0)
        wr = comp_ref[r, 0] * bases_ref[0]
        for b in range(1, NB):
            wr = wr + comp_ref[r, b] * bases_ref[b]
        acc = acc + jnp.dot(a * rec, wr, preferred_element_type=jnp.float32)
    deg = jnp.sum(cnt[0:C, 0:R], axis=1, keepdims=True)
    t = tagg_ref[0, 0:C, :] / jnp.maximum(deg, 1.0)
    out = acc + bias_ref[...] + jnp.dot(t, wtpT_ref[...],
                                        preferred_element_type=jnp.float32)
    out = out + btp_ref[...]
    out = jnp.maximum(out, 0.0)
    mu = jnp.mean(out, axis=1, keepdims=True)
    var = jnp.mean((out - mu) ** 2, axis=1, keepdims=True)
    out_ref[...] = (out - mu) * lax.rsqrt(var + 1e-5) * gamma_ref[...] \
        + beta_ref[...]


_dense_kernel = pl.pallas_call(
    _dense_body,
    grid=(NCH,),
    in_specs=[
        pl.BlockSpec((1, R * CP, D), lambda k: (k, 0, 0)),
        pl.BlockSpec((1, CP, D), lambda k: (k, 0, 0)),
        pl.BlockSpec((1, CP, D), lambda k: (k, 0, 0)),
        pl.BlockSpec((C, D), lambda k: (k, 0)),
        pl.BlockSpec((R, NB), lambda k: (0, 0)),
        pl.BlockSpec((NB, D, D), lambda k: (0, 0, 0)),
        pl.BlockSpec((D, D), lambda k: (0, 0)),
        pl.BlockSpec((1, D), lambda k: (0, 0)),
        pl.BlockSpec((D, D), lambda k: (0, 0)),
        pl.BlockSpec((1, D), lambda k: (0, 0)),
        pl.BlockSpec((1, D), lambda k: (0, 0)),
        pl.BlockSpec((1, D), lambda k: (0, 0)),
    ],
    out_specs=pl.BlockSpec((C, D), lambda k: (k, 0)),
    out_shape=jax.ShapeDtypeStruct((N, D), jnp.float32),
)


def kernel(x, edge_index, edge_type, edge_time, comp, bases, root, bias,
           W_tp, b_tp, gamma, beta):
    w = _w_kernel(edge_time.reshape(E // D, D)).reshape(E)
    w_i = jax.lax.bitcast_convert_type(w, jnp.int32)
    meta = jnp.stack([edge_index[0].reshape(NS, ET // BE, BE),
                      edge_index[1].reshape(NS, ET // BE, BE),
                      edge_type.reshape(NS, ET // BE, BE),
                      w_i.reshape(NS, ET // BE, BE)], axis=2)
    xp = jnp.zeros((XP, D), jnp.float32).at[:N].set(x)
    z = jnp.zeros((8192,), jnp.float32)

    agg_raw, tagg_raw, cnt_raw = _get_sc_kernel()(meta, xp, z)
    cnt3 = cnt_raw.reshape(NCH, CP, D)

    out = _dense_kernel(agg_raw, cnt3, tagg_raw, x, comp, bases, root,
                        bias.reshape(1, D), W_tp.T, b_tp.reshape(1, D),
                        gamma.reshape(1, D), beta.reshape(1, D))
    return out


# speedup vs baseline: 1.4450x; 1.0041x over previous
import functools

import jax
import jax.numpy as jnp
from jax import lax
from jax.experimental import pallas as pl
from jax.experimental.pallas import tpu as pltpu
from jax.experimental.pallas import tpu_sc as plsc

N = 10000
E = 320000
D = 128
R = 8
NB = 4

NC = 2
NS = 16
C = 1000
CP = 1024
NCH = 10
CPS = NCH // NC
ET = E // NS
NV = ET // 16
CAP = 128
XP = 20480
BE = 4000
THR = CAP - 16


def _w_body(t_ref, w_ref):
    t = t_ref[...]
    ct = jnp.max(t)
    td = ct - t
    td = td / (jnp.max(td) + 1e-8)
    w = jnp.exp(-0.1 * td)
    w_ref[...] = w / (jnp.sum(w) + 1e-8)


_w_kernel = pl.pallas_call(
    _w_body,
    out_shape=jax.ShapeDtypeStruct((E // D, D), jnp.float32),
)


def _sc_body(meta_hbm, x_hbm, z_hbm,
             agg_out, tagg_out, cnt_out,
             mq, rows_v,
             pk_buf, src_buf, loc_buf, cloc_buf, dstl_buf, w_buf, val_buf,
             agg_sp, tagg_sp, cnt_sp, sem):
    cid = lax.axis_index("c")
    sid = lax.axis_index("s")

    zrow = 10240 + sid * 16
    zero16f = jnp.zeros((16,), jnp.float32)
    zero16i = jnp.zeros((16,), jnp.int32)
    zrowv = jnp.full((16,), zrow, jnp.int32)

    one16f = jnp.ones((16,), jnp.float32)

    def reset_bufs():
        for t in range(CAP // 16):
            sl = pl.ds(t * 16, 16)
            pk_buf[sl] = zrowv
            w_buf[sl] = zero16f

    reset_bufs()

    s512 = pl.multiple_of(sid * 512, 512)
    s64 = pl.multiple_of(sid * 64, 64)
    s8k = pl.multiple_of(sid * 8192, 8192)

    def flush():
        for t in range(CAP // 16):
            sl = pl.ds(t * 16, 16)
            pk = pk_buf[sl]
            vsrc = pk & 16383
            dl = (pk >> 14) & 1023
            vtyp = pk >> 24
            src_buf[sl] = vsrc
            loc_buf[sl] = vtyp * CP + dl
            cloc_buf[sl] = dl * D + vtyp
            dstl_buf[sl] = dl
            val_buf[sl] = jnp.where(vsrc < N, one16f, zero16f)
        pltpu.async_copy(x_hbm.at[src_buf], rows_v, sem).wait()
        pltpu.sync_copy(rows_v, agg_sp.at[loc_buf], add=True)
        pltpu.sync_copy(val_buf, cnt_sp.at[cloc_buf], add=True)
        def rb(j, c):
            wv = plsc.load_gather(w_buf, [jnp.full((16,), j, jnp.int32)])
            for g in range(8):
                sl = pl.ds(g * 16, 16)
                rows_v[j, sl] = rows_v[j, sl] * wv
            return c
        lax.fori_loop(0, CAP, rb, 0, unroll=8)
        pltpu.sync_copy(rows_v, tagg_sp.at[dstl_buf], add=True)
        reset_bufs()

    def chunk_body(i, _carry):
        k = cid * CPS + i
        c0 = k * C

        pltpu.sync_copy(x_hbm.at[pl.ds(10240, 512)],
                        agg_sp.at[pl.ds(s512, 512)])
        pltpu.sync_copy(x_hbm.at[pl.ds(10240, 64)],
                        tagg_sp.at[pl.ds(s64, 64)])
        pltpu.sync_copy(z_hbm, cnt_sp.at[pl.ds(s8k, 8192)])
        plsc.subcore_barrier()

        def block_body(q, n_acc0):
            pltpu.sync_copy(meta_hbm.at[sid, q], mq)

            def vloop(v, n_acc):
                esl = pl.ds(v * 16, 16)
                vdst = mq[1, esl]
                vsrc = mq[0, esl]
                vtyp = mq[2, esl]
                vw = plsc.bitcast(mq[3, esl], jnp.float32)
                dl = vdst - c0
                m = (dl >= 0) & (dl < C)
                pk = vsrc + (dl << 14) + (vtyp << 24)
                cum = plsc.cumsum(m.astype(jnp.int32))
                pos = n_acc + cum - 1
                plsc.store_scatter(pk_buf, [pos], pk, mask=m)
                plsc.store_scatter(w_buf, [pos], vw, mask=m)
                n_acc = n_acc + plsc.all_reduce_population_count(m)[0]

                @pl.when(n_acc > THR)
                def _():
                    flush()

                return jnp.where(n_acc > THR, 0, n_acc)

            return lax.fori_loop(0, BE // 16, vloop, n_acc0, unroll=4)

        lax.fori_loop(0, ET // BE, block_body, 0)
        flush()

        plsc.subcore_barrier()
        pltpu.sync_copy(agg_sp.at[pl.ds(s512, 512)],
                        agg_out.at[k, pl.ds(s512, 512)])
        pltpu.sync_copy(tagg_sp.at[pl.ds(s64, 64)],
                        tagg_out.at[k, pl.ds(s64, 64)])
        pltpu.sync_copy(cnt_sp.at[pl.ds(s8k, 8192)],
                        cnt_out.at[k, pl.ds(s8k, 8192)])
        plsc.subcore_barrier()
        return 0

    lax.fori_loop(0, CPS, chunk_body, 0)


@functools.cache
def _get_sc_kernel():
    return functools.partial(
        pl.kernel,
        out_type=(
            jax.ShapeDtypeStruct((NCH, R * CP, D), jnp.float32),
            jax.ShapeDtypeStruct((NCH, CP, D), jnp.float32),
            jax.ShapeDtypeStruct((NCH, CP * D), jnp.float32),
        ),
        mesh=plsc.VectorSubcoreMesh(core_axis_name="c", subcore_axis_name="s"),
        compiler_params=pltpu.CompilerParams(needs_layout_passes=False, use_tc_tiling_on_sc=False),
        scratch_types=[
            pltpu.VMEM((4, BE), jnp.int32),
            pltpu.VMEM((CAP, D), jnp.float32),
            pltpu.VMEM((CAP,), jnp.int32),
            pltpu.VMEM((CAP,), jnp.int32),
            pltpu.VMEM((CAP,), jnp.int32),
            pltpu.VMEM((CAP,), jnp.int32),
            pltpu.VMEM((CAP,), jnp.int32),
            pltpu.VMEM((CAP,), jnp.float32),
            pltpu.VMEM((CAP,), jnp.float32),
            pltpu.VMEM_SHARED((R * CP, D), jnp.float32),
            pltpu.VMEM_SHARED((CP, D), jnp.float32),
            pltpu.VMEM_SHARED((CP * D,), jnp.float32),
            pltpu.SemaphoreType.DMA,
        ],
    )(_sc_body)


def _dense_body(agg_ref, cnt_ref, tagg_ref, x_ref, comp_ref, bases_ref,
                root_ref, bias_ref, wtpT_ref, btp_ref, gamma_ref, beta_ref,
                out_ref):
    xb = x_ref[...]
    cnt = cnt_ref[0]
    acc = jnp.dot(xb, root_ref[...], preferred_element_type=jnp.float32)
    for r in range(R):
        a = agg_ref[0, pl.ds(r * CP, C), :]
        rec = 1.0 / jnp.maximum(cnt[0:C, r:r + 1], 1.0)
        wr = comp_ref[r, 0] * bases_ref[0]
        for b in range(1, NB):
            wr = wr + comp_ref[r, b] * bases_ref[b]
        acc = acc + jnp.dot(a * rec, wr, preferred_element_type=jnp.float32)
    deg = jnp.sum(cnt[0:C, 0:R], axis=1, keepdims=True)
    t = tagg_ref[0, 0:C, :] / jnp.maximum(deg, 1.0)
    out = acc + bias_ref[...] + jnp.dot(t, wtpT_ref[...],
                                        preferred_element_type=jnp.float32)
    out = out + btp_ref[...]
    out = jnp.maximum(out, 0.0)
    mu = jnp.mean(out, axis=1, keepdims=True)
    var = jnp.mean((out - mu) ** 2, axis=1, keepdims=True)
    out_ref[...] = (out - mu) * lax.rsqrt(var + 1e-5) * gamma_ref[...] \
        + beta_ref[...]


_dense_kernel = pl.pallas_call(
    _dense_body,
    grid=(NCH,),
    in_specs=[
        pl.BlockSpec((1, R * CP, D), lambda k: (k, 0, 0)),
        pl.BlockSpec((1, CP, D), lambda k: (k, 0, 0)),
        pl.BlockSpec((1, CP, D), lambda k: (k, 0, 0)),
        pl.BlockSpec((C, D), lambda k: (k, 0)),
        pl.BlockSpec((R, NB), lambda k: (0, 0)),
        pl.BlockSpec((NB, D, D), lambda k: (0, 0, 0)),
        pl.BlockSpec((D, D), lambda k: (0, 0)),
        pl.BlockSpec((1, D), lambda k: (0, 0)),
        pl.BlockSpec((D, D), lambda k: (0, 0)),
        pl.BlockSpec((1, D), lambda k: (0, 0)),
        pl.BlockSpec((1, D), lambda k: (0, 0)),
        pl.BlockSpec((1, D), lambda k: (0, 0)),
    ],
    out_specs=pl.BlockSpec((C, D), lambda k: (k, 0)),
    out_shape=jax.ShapeDtypeStruct((N, D), jnp.float32),
)


def kernel(x, edge_index, edge_type, edge_time, comp, bases, root, bias,
           W_tp, b_tp, gamma, beta):
    w = _w_kernel(edge_time.reshape(E // D, D)).reshape(E)
    w_i = jax.lax.bitcast_convert_type(w, jnp.int32)
    meta = jnp.stack([edge_index[0].reshape(NS, ET // BE, BE),
                      edge_index[1].reshape(NS, ET // BE, BE),
                      edge_type.reshape(NS, ET // BE, BE),
                      w_i.reshape(NS, ET // BE, BE)], axis=2)
    xp = jnp.zeros((XP, D), jnp.float32).at[:N].set(x)
    z = jnp.zeros((8192,), jnp.float32)

    agg_raw, tagg_raw, cnt_raw = _get_sc_kernel()(meta, xp, z)
    cnt3 = cnt_raw.reshape(NCH, CP, D)

    out = _dense_kernel(agg_raw, cnt3, tagg_raw, x, comp, bases, root,
                        bias.reshape(1, D), W_tp.T, b_tp.reshape(1, D),
                        gamma.reshape(1, D), beta.reshape(1, D))
    return out
